# baseline (speedup 1.0000x reference)
import functools

import jax
import jax.numpy as jnp
from jax import lax
from jax.experimental import pallas as pl
from jax.experimental.pallas import tpu as pltpu
from jax.experimental.pallas import tpu_sc as plsc


def _fuse_body(tablet_ref, w_ref, b_ref, g_ref, be_ref, out_ref):
    h = lax.dot_general(
        tablet_ref[...],
        w_ref[...],
        dimension_numbers=(((0,), (0,)), ((), ())),
        preferred_element_type=jnp.float32,
    )
    h = h + b_ref[...]
    mean = jnp.mean(h, axis=-1, keepdims=True)
    var = jnp.mean((h - mean) ** 2, axis=-1, keepdims=True)
    h = (h - mean) * lax.rsqrt(var + 1e-5)
    h = h * g_ref[...] + be_ref[...]
    out_ref[...] = jnp.maximum(h, 0.0)


def _fuse_table(table, W, b, gamma, beta, block_v=12288):
    V, D = table.shape
    E = W.shape[1]
    tablet = table.T
    grid = (pl.cdiv(V, block_v),)
    return pl.pallas_call(
        _fuse_body,
        grid=grid,
        in_specs=[
            pl.BlockSpec((D, block_v), lambda i: (0, i)),
            pl.BlockSpec((D, E), lambda i: (0, 0)),
            pl.BlockSpec((1, E), lambda i: (0, 0)),
            pl.BlockSpec((1, E), lambda i: (0, 0)),
            pl.BlockSpec((1, E), lambda i: (0, 0)),
        ],
        out_specs=pl.BlockSpec((block_v, E), lambda i: (i, 0)),
        out_shape=jax.ShapeDtypeStruct((V, E), jnp.float32),
    )(tablet, W, b.reshape(1, E), gamma.reshape(1, E), beta.reshape(1, E))


def _make_gather(E, n_total, chunk=128, nbuf=5):
    info = plsc.get_sparse_core_info()
    nc, ns = info.num_cores, info.num_subcores
    nw = nc * ns
    n_chunks = n_total // (nw * chunk)
    assert n_chunks * nw * chunk == n_total and n_chunks % nbuf == 0
    n_groups = n_chunks // nbuf
    mesh = plsc.VectorSubcoreMesh(core_axis_name="c", subcore_axis_name="s")

    @functools.partial(
        pl.kernel,
        mesh=mesh,
        out_type=jax.ShapeDtypeStruct((n_total, E), jnp.float32),
        scratch_types=[
            pltpu.VMEM((n_chunks, chunk), jnp.int32),
            pltpu.VMEM((nbuf, chunk, E), jnp.float32),
        ]
        + [pltpu.SemaphoreType.DMA] * (2 * nbuf),
    )
    def gather(table_hbm, idx_hbm, out_hbm, idx_v, rows_v, *sems):
        gsems, ssems = sems[:nbuf], sems[nbuf:]
        wid = lax.axis_index("s") * nc + lax.axis_index("c")
        pltpu.sync_copy(idx_hbm.at[wid], idx_v)

        def out_at(c):
            return out_hbm.at[pl.ds((wid * n_chunks + c) * chunk, chunk)]

        for j in range(nbuf):
            pltpu.async_copy(table_hbm.at[idx_v.at[j]], rows_v.at[j], gsems[j])

        def body(g, carry):
            for j in range(nbuf):
                c = g * nbuf + j
                pltpu.make_async_copy(
                    table_hbm.at[idx_v.at[c]], rows_v.at[j], gsems[j]
                ).wait()
                pltpu.async_copy(rows_v.at[j], out_at(c), ssems[j])
            for j in range(nbuf):
                c_next = (g + 1) * nbuf + j

                @pl.when(c_next < n_chunks)
                def _():
                    pltpu.make_async_copy(
                        rows_v.at[j], out_at(c_next - nbuf), ssems[j]
                    ).wait()
                    pltpu.async_copy(
                        table_hbm.at[idx_v.at[c_next]], rows_v.at[j], gsems[j]
                    )

            return carry

        lax.fori_loop(0, n_groups, body, 0)
        for j in range(nbuf):
            c = (n_groups - 1) * nbuf + j
            pltpu.make_async_copy(rows_v.at[j], out_at(c), ssems[j]).wait()

    def run(fused, idx_flat):
        idx3 = idx_flat.reshape(nw, n_chunks, chunk)
        return gather(fused, idx3)

    return run


def kernel(token_ids, table, W, b, gamma, beta):
    B, S = token_ids.shape
    E = W.shape[1]
    fused = _fuse_table(table, W, b, gamma, beta)
    idx_flat = token_ids.T.reshape(-1).astype(jnp.int32)
    out = _make_gather(E, B * S)(fused, idx_flat)
    return out.reshape(S, B, E).transpose(1, 0, 2)


# speedup vs baseline: 1.0284x; 1.0284x over previous
import functools

import jax
import jax.numpy as jnp
from jax import lax
from jax.experimental import pallas as pl
from jax.experimental.pallas import tpu as pltpu
from jax.experimental.pallas import tpu_sc as plsc


def _fuse_body(tablet_ref, w_ref, b_ref, g_ref, be_ref, out_ref):
    h = lax.dot_general(
        tablet_ref[...],
        w_ref[...],
        dimension_numbers=(((0,), (0,)), ((), ())),
        preferred_element_type=jnp.float32,
    )
    h = h + b_ref[...]
    mean = jnp.mean(h, axis=-1, keepdims=True)
    var = jnp.mean((h - mean) ** 2, axis=-1, keepdims=True)
    h = (h - mean) * lax.rsqrt(var + 1e-5)
    h = h * g_ref[...] + be_ref[...]
    out_ref[...] = jnp.maximum(h, 0.0)


def _fuse_table(table, W, b, gamma, beta, block_v=8192):
    V, D = table.shape
    E = W.shape[1]
    tablet = table.T
    grid = (pl.cdiv(V, block_v),)
    return pl.pallas_call(
        _fuse_body,
        grid=grid,
        in_specs=[
            pl.BlockSpec((D, block_v), lambda i: (0, i)),
            pl.BlockSpec((D, E), lambda i: (0, 0)),
            pl.BlockSpec((1, E), lambda i: (0, 0)),
            pl.BlockSpec((1, E), lambda i: (0, 0)),
            pl.BlockSpec((1, E), lambda i: (0, 0)),
        ],
        out_specs=pl.BlockSpec((block_v, E), lambda i: (i, 0)),
        out_shape=jax.ShapeDtypeStruct((V, E), jnp.float32),
    )(tablet, W, b.reshape(1, E), gamma.reshape(1, E), beta.reshape(1, E))


def _make_gather(E, n_total, chunk=128, nbuf=3):
    info = plsc.get_sparse_core_info()
    nc, ns = info.num_cores, info.num_subcores
    nw = nc * ns
    n_chunks = n_total // (nw * chunk)
    assert n_chunks * nw * chunk == n_total and n_chunks % 2 == 0
    n_super = n_chunks // 2
    mesh = plsc.VectorSubcoreMesh(core_axis_name="c", subcore_axis_name="s")

    @functools.partial(
        pl.kernel,
        mesh=mesh,
        out_type=jax.ShapeDtypeStruct((n_total, E), jnp.float32),
        scratch_types=[
            pltpu.VMEM((n_chunks, chunk), jnp.int32),
            pltpu.VMEM((nbuf, 2 * chunk, E), jnp.float32),
        ]
        + [pltpu.SemaphoreType.DMA] * (2 * nbuf),
    )
    def gather(table_hbm, idx_hbm, out_hbm, idx_v, rows_v, *sems):
        gsems, ssems = sems[:nbuf], sems[nbuf:]
        wid = lax.axis_index("s") * nc + lax.axis_index("c")
        pltpu.sync_copy(idx_hbm.at[wid], idx_v)

        def gat(s, j, h):
            return pltpu.make_async_copy(
                table_hbm.at[idx_v.at[2 * s + h]],
                rows_v.at[j, pl.ds(h * chunk, chunk)],
                gsems[j],
            )

        def sca(s, j):
            return pltpu.make_async_copy(
                rows_v.at[j],
                out_hbm.at[pl.ds((wid * n_super + s) * 2 * chunk, 2 * chunk)],
                ssems[j],
            )

        for j in range(nbuf):
            if j < n_super:
                gat(j, j, 0).start()
                gat(j, j, 1).start()

        def mbody(g, carry):
            for j in range(nbuf):
                s = g * nbuf + j

                @pl.when(s < n_super)
                def _(s=s, j=j):
                    gat(s, j, 0).wait()
                    gat(s, j, 1).wait()
                    sca(s, j).start()

                    @pl.when(s + nbuf < n_super)
                    def _(s=s, j=j):
                        sca(s, j).wait()
                        gat(s + nbuf, j, 0).start()
                        gat(s + nbuf, j, 1).start()

            return carry

        lax.fori_loop(0, (n_super + nbuf - 1) // nbuf, mbody, 0)

        for j in range(nbuf):
            if j < n_super:
                sca(0, j).wait()

    def run(fused, idx_flat):
        idx3 = idx_flat.reshape(nw, n_chunks, chunk)
        return gather(fused, idx3)

    return run


def kernel(token_ids, table, W, b, gamma, beta):
    B, S = token_ids.shape
    E = W.shape[1]
    fused = _fuse_table(table, W, b, gamma, beta)
    idx_flat = token_ids.T.reshape(-1).astype(jnp.int32)
    out = _make_gather(E, B * S)(fused, idx_flat)
    return out.reshape(S, B, E).transpose(1, 0, 2)
